# Initial kernel scaffold; baseline (speedup 1.0000x reference)
import jax
import jax.numpy as jnp
from jax.experimental import pallas as pl


def kernel(x, edge_index, W, a):
    raise NotImplementedError("write your pallas kernel here")


# speedup vs baseline: 6.4622x; 6.4622x over previous
import functools

import jax
import jax.numpy as jnp
from jax import lax
from jax.experimental import pallas as pl
from jax.experimental.pallas import tpu as pltpu
from jax.experimental.pallas import tpu_sc as plsc

N = 10000
E = 160000
DIN = 256
DOUT = 256
ALPHA = 0.2

HALF = 128
ACCW = 144
EB = 128
NSUB = 16
CHUNK_B = 79
CHUNK = CHUNK_B * EB
E_PAD = NSUB * CHUNK
ROWS_PER_SUB = 632
N_ACC = NSUB * ROWS_PER_SUB
_ZCHUNKS = ((0, 128), (128, 128), (256, 128), (384, 128), (512, 120))


def _mm_body(x_ref, w_ref, a2_ref, wh_ref, s2_ref):
    wh = lax.dot_general(x_ref[...], w_ref[...], (((1,), (1,)), ((), ())),
                         preferred_element_type=jnp.float32)
    wh_ref[0] = wh[:, :HALF]
    wh_ref[1] = wh[:, HALF:]
    s2_ref[...] = lax.dot_general(wh, a2_ref[...], (((1,), (0,)), ((), ())),
                                  preferred_element_type=jnp.float32)


def _matmul(x, W, a2):
    BN = 1000
    grid = N // BN
    return pl.pallas_call(
        _mm_body,
        grid=(grid,),
        in_specs=[
            pl.BlockSpec((BN, DIN), lambda i: (i, 0)),
            pl.BlockSpec((DOUT, DIN), lambda i: (0, 0)),
            pl.BlockSpec((DIN, 2), lambda i: (0, 0)),
        ],
        out_specs=[
            pl.BlockSpec((2, BN, HALF), lambda i: (0, i, 0)),
            pl.BlockSpec((BN, 2), lambda i: (i, 0)),
        ],
        out_shape=[
            jax.ShapeDtypeStruct((2, N, HALF), jnp.float32),
            jax.ShapeDtypeStruct((N, 2), jnp.float32),
        ],
    )(x, W, a2)


def _sc_call(row_p, col_p, s_i, s_j, wh_cat):
    mesh = plsc.VectorSubcoreMesh(core_axis_name="c", subcore_axis_name="s")

    @functools.partial(
        pl.kernel,
        mesh=mesh,
        compiler_params=pltpu.CompilerParams(needs_layout_passes=False,
                                             use_tc_tiling_on_sc=False),
        out_type=[
            jax.ShapeDtypeStruct((2, N_ACC, HALF), jnp.float32),
            jax.ShapeDtypeStruct((N_ACC, 16), jnp.float32),
        ],
        scratch_types=[
            pltpu.VMEM((N,), jnp.float32),
            pltpu.VMEM((N,), jnp.float32),
            pltpu.VMEM((EB,), jnp.int32),
            pltpu.VMEM((EB,), jnp.int32),
            pltpu.VMEM((EB,), jnp.float32),
            pltpu.VMEM((EB, HALF), jnp.float32),
            pltpu.VMEM((EB, 16), jnp.float32),
            pltpu.VMEM_SHARED((N_ACC, HALF), jnp.float32),
            pltpu.VMEM_SHARED((N_ACC, 16), jnp.float32),
            pltpu.SemaphoreType.DMA,
        ],
    )
    def sc_kernel(row_hbm, col_hbm, si_hbm, sj_hbm, whcat_hbm,
                  out_hbm, outd_hbm,
                  si_v, sj_v, rowb_v, colb_v, wb_v, gbuf_v, wbuf_v,
                  acc_sh, accd_sh, gsem):
        cid = lax.axis_index("c")
        sid = lax.axis_index("s")
        lane = lax.iota(jnp.int32, 16)
        zeros16 = jnp.zeros((16,), jnp.float32)

        pltpu.sync_copy(si_hbm, si_v)
        pltpu.sync_copy(sj_hbm, sj_v)

        def _zrow(r, _):
            for k in range(HALF // 16):
                gbuf_v[r, pl.ds(k * 16, 16)] = zeros16
            wbuf_v[r, pl.ds(0, 16)] = zeros16
            return 0
        lax.fori_loop(0, EB, _zrow, 0)
        zbase = sid * ROWS_PER_SUB
        for off, nrows in _ZCHUNKS:
            pltpu.sync_copy(gbuf_v.at[pl.ds(0, nrows)],
                            acc_sh.at[pl.ds(zbase + off, nrows)])
            pltpu.sync_copy(wbuf_v.at[pl.ds(0, nrows)],
                            accd_sh.at[pl.ds(zbase + off, nrows)])
        plsc.subcore_barrier()

        col_off = cid * N

        def _batch(b, _):
            start = sid * CHUNK + b * EB
            pltpu.sync_copy(row_hbm.at[pl.ds(start, EB)], rowb_v)
            pltpu.sync_copy(col_hbm.at[pl.ds(start, EB)], colb_v)
            for k in range(EB // 16):
                off = k * 16
                rv = rowb_v[pl.ds(off, 16)]
                cv = colb_v[pl.ds(off, 16)]
                e = plsc.load_gather(si_v, [rv]) + plsc.load_gather(sj_v, [cv])
                e = jnp.where(e > 0, e, ALPHA * e)
                w = jnp.exp(e)
                gi = start + off + lane
                w = jnp.where(gi < E, w, 0.0)
                wb_v[pl.ds(off, 16)] = w
                plsc.store_scatter(wbuf_v, [off + lane, lane * 0], w)
                colb_v[pl.ds(off, 16)] = cv + col_off
            pltpu.async_copy(whcat_hbm.at[colb_v], gbuf_v, gsem).wait()

            def _srow(r, _):
                wspl = plsc.load_gather(wb_v, [jnp.full((16,), r, jnp.int32)])
                for k in range(HALF // 16):
                    gbuf_v[r, pl.ds(k * 16, 16)] = (
                        gbuf_v[r, pl.ds(k * 16, 16)] * wspl)
                return 0
            lax.fori_loop(0, EB, _srow, 0)

            pltpu.sync_copy(gbuf_v, acc_sh.at[rowb_v], add=True)
            pltpu.sync_copy(wbuf_v, accd_sh.at[rowb_v], add=True)
            return 0

        lax.fori_loop(0, CHUNK_B, _batch, 0)
        plsc.subcore_barrier()

        wbase = sid * ROWS_PER_SUB
        for off, nrows in _ZCHUNKS:
            pltpu.sync_copy(acc_sh.at[pl.ds(wbase + off, nrows)],
                            out_hbm.at[cid, pl.ds(wbase + off, nrows)])

        @pl.when(cid == 0)
        def _():
            for off, nrows in _ZCHUNKS:
                pltpu.sync_copy(accd_sh.at[pl.ds(wbase + off, nrows)],
                                outd_hbm.at[pl.ds(wbase + off, nrows)])

    return sc_kernel(row_p, col_p, s_i, s_j, wh_cat)


def _elu_body(acc0_ref, acc1_ref, den_ref, out_ref):
    num = jnp.concatenate([acc0_ref[...], acc1_ref[...]], axis=1)
    den = den_ref[:, 0:1]
    pos = den > 0
    z = jnp.where(pos, num / jnp.where(pos, den, 1.0), 0.0)
    out_ref[...] = jnp.where(z > 0, z, jnp.exp(z) - 1.0)


def _epilogue(acc, den):
    BN = 1000
    grid = N // BN
    return pl.pallas_call(
        _elu_body,
        grid=(grid,),
        in_specs=[
            pl.BlockSpec((BN, HALF), lambda i: (i, 0)),
            pl.BlockSpec((BN, HALF), lambda i: (i, 0)),
            pl.BlockSpec((BN, 16), lambda i: (i, 0)),
        ],
        out_specs=pl.BlockSpec((BN, DOUT), lambda i: (i, 0)),
        out_shape=jax.ShapeDtypeStruct((N, DOUT), jnp.float32),
    )(acc[0], acc[1], den)


def kernel(x, edge_index, W, a):
    a2 = jnp.stack([a[0, :DOUT], a[0, DOUT:]], axis=1)
    wh_halves, s2 = _matmul(x, W, a2)
    wh_cat = wh_halves.reshape(2 * N, HALF)
    s_i = s2[:, 0]
    s_j = s2[:, 1]
    row = jnp.pad(edge_index[0], (0, E_PAD - E))
    col = jnp.pad(edge_index[1], (0, E_PAD - E))
    acc, den = _sc_call(row, col, s_i, s_j, wh_cat)
    return _epilogue(acc[:, :N, :], den[:N])
